# baseline (speedup 1.0000x reference)
import functools

import jax
import jax.numpy as jnp
from jax import lax
from jax.experimental import pallas as pl
from jax.experimental.pallas import tpu as pltpu
from jax.experimental.pallas import tpu_sc as plsc

B, K, C, H, W = 4, 16, 3, 384, 384
P = H * W
HB = 64
NW = 32
PPW = P // NW
CH = PPW // 2
NCHUNK = PPW // CH
LANES = 16


def _blend_block(scores_ref, cand_ref, mask_ref, partial_ref, final_ref):
    scores = scores_ref[0]
    best = jnp.argmax(scores, axis=0).astype(jnp.int32)
    vis = mask_ref[0, 0]
    fill = 1.0 - vis
    sel = [None, None, None]
    for k in range(K):
        onehot = best == k
        for c in range(C):
            pix = jnp.where(onehot, cand_ref[0, k, c], 0.0)
            sel[c] = pix if sel[c] is None else sel[c] + pix
    for c in range(C):
        final_ref[0, c] = partial_ref[0, c] * vis + sel[c] * fill


def _final_image(candidate_images, selection_scores, mask, partial_image):
    return pl.pallas_call(
        _blend_block,
        grid=(B, H // HB),
        in_specs=[
            pl.BlockSpec((1, K, HB, W), lambda b, h: (b, 0, h, 0)),
            pl.BlockSpec((1, K, C, HB, W), lambda b, h: (b, 0, 0, h, 0)),
            pl.BlockSpec((1, 1, HB, W), lambda b, h: (b, 0, h, 0)),
            pl.BlockSpec((1, C, HB, W), lambda b, h: (b, 0, h, 0)),
        ],
        out_specs=pl.BlockSpec((1, C, HB, W), lambda b, h: (b, 0, h, 0)),
        out_shape=jax.ShapeDtypeStruct((B, C, H, W), jnp.float32),
    )(selection_scores, candidate_images, mask, partial_image)


@functools.partial(
    pl.kernel,
    mesh=plsc.VectorSubcoreMesh(core_axis_name="c", subcore_axis_name="s"),
    out_type=jax.ShapeDtypeStruct((B, K, P), jnp.float32),
    scratch_types=[
        pltpu.VMEM((K, CH), jnp.float32),
        pltpu.VMEM((K, CH), jnp.float32),
        pltpu.SemaphoreType.DMA,
        pltpu.SemaphoreType.DMA,
    ],
)
def _weights_sc(scores_hbm, out_hbm, in_v, out_v, in_sem, out_sem):
    num_cores = 2
    wid = lax.axis_index("s") * num_cores + lax.axis_index("c")

    def compute_chunk(_, unused):
        for k in range(K):
            s_k = in_v[k, pl.ds(_ * LANES, LANES)]
            if k == 0:
                s = [s_k]
                m = s_k
            else:
                s.append(s_k)
                m = jnp.maximum(m, s_k)
        notfound = jnp.ones((LANES,), jnp.float32)
        for k in range(K):
            is_max = jnp.where(s[k] == m, 1.0, 0.0)
            out_v[k, pl.ds(_ * LANES, LANES)] = is_max * notfound
            notfound = notfound * (1.0 - is_max)
        return unused

    def run_chunk(i, unused):
        b = i // NCHUNK
        cidx = i - b * NCHUNK
        p0 = wid * PPW + cidx * CH
        in_copies = [
            pltpu.async_copy(
                scores_hbm.at[b, k, pl.ds(p0, CH)], in_v.at[k], in_sem)
            for k in range(K)
        ]
        for cp in in_copies:
            cp.wait()
        lax.fori_loop(0, CH // LANES, compute_chunk, 0, unroll=False)
        out_copies = [
            pltpu.async_copy(
                out_v.at[k], out_hbm.at[b, k, pl.ds(p0, CH)], out_sem)
            for k in range(K)
        ]
        for cp in out_copies:
            cp.wait()
        return unused

    lax.fori_loop(0, B * NCHUNK, run_chunk, 0, unroll=False)


def kernel(candidate_images, selection_scores, mask, partial_image):
    final_image = _final_image(
        candidate_images, selection_scores, mask, partial_image)
    weights = _weights_sc(selection_scores.reshape(B, K, P))
    return (final_image, weights.reshape(B, K, H, W))


# speedup vs baseline: 2.2167x; 2.2167x over previous
import functools

import jax
import jax.numpy as jnp
from jax import lax
from jax.experimental import pallas as pl
from jax.experimental.pallas import tpu as pltpu
from jax.experimental.pallas import tpu_sc as plsc

B, K, C, H, W = 4, 16, 3, 384, 384
HB = 64
LANES = 16
TR, TCOL = 8, 128
TILES_H = H // TR
TILES_W = W // TCOL
TILES_PER_B = TILES_H * TILES_W
NTILES = B * TILES_PER_B
NW = 32
CPW = NTILES // NW
GROUPS = TR * TCOL // LANES


def _blend_block(scores_ref, cand_ref, mask_ref, partial_ref, final_ref):
    scores = scores_ref[0]
    best = jnp.argmax(scores, axis=0).astype(jnp.int32)
    vis = mask_ref[0, 0]
    fill = 1.0 - vis
    sel = [None, None, None]
    for k in range(K):
        onehot = best == k
        for c in range(C):
            pix = jnp.where(onehot, cand_ref[0, k, c], 0.0)
            sel[c] = pix if sel[c] is None else sel[c] + pix
    for c in range(C):
        final_ref[0, c] = partial_ref[0, c] * vis + sel[c] * fill


def _final_image(candidate_images, selection_scores, mask, partial_image):
    return pl.pallas_call(
        _blend_block,
        grid=(B, H // HB),
        in_specs=[
            pl.BlockSpec((1, K, HB, W), lambda b, h: (b, 0, h, 0)),
            pl.BlockSpec((1, K, C, HB, W), lambda b, h: (b, 0, 0, h, 0)),
            pl.BlockSpec((1, 1, HB, W), lambda b, h: (b, 0, h, 0)),
            pl.BlockSpec((1, C, HB, W), lambda b, h: (b, 0, h, 0)),
        ],
        out_specs=pl.BlockSpec((1, C, HB, W), lambda b, h: (b, 0, h, 0)),
        out_shape=jax.ShapeDtypeStruct((B, C, H, W), jnp.float32),
    )(selection_scores, candidate_images, mask, partial_image)


@functools.partial(
    pl.kernel,
    mesh=plsc.VectorSubcoreMesh(core_axis_name="c", subcore_axis_name="s"),
    out_type=jax.ShapeDtypeStruct((B, K, H, W), jnp.float32),
    scratch_types=[
        pltpu.VMEM((K, TR, TCOL), jnp.float32),
        pltpu.VMEM((K, TR, TCOL), jnp.float32),
        pltpu.VMEM((K, TR, TCOL), jnp.float32),
        pltpu.VMEM((K, TR, TCOL), jnp.float32),
        pltpu.SemaphoreType.DMA,
        pltpu.SemaphoreType.DMA,
        pltpu.SemaphoreType.DMA,
        pltpu.SemaphoreType.DMA,
    ],
    compiler_params=pltpu.CompilerParams(use_tc_tiling_on_sc=True),
)
def _weights_sc(scores_hbm, out_hbm, in0, in1, out0, out1,
                si0, si1, so0, so1):
    num_cores = 2
    wid = lax.axis_index("s") * num_cores + lax.axis_index("c")
    w0 = wid * CPW

    def chunk_slices(j):
        t = w0 + j
        b = t // TILES_PER_B
        rem = t - b * TILES_PER_B
        r = rem // TILES_W
        tw = rem - r * TILES_W
        return b, pl.ds(r * TR, TR), pl.ds(tw * TCOL, TCOL)

    def fire_in(j, buf, sem):
        b, rs, cs = chunk_slices(j)
        for k in range(K):
            pltpu.async_copy(scores_hbm.at[b, k, rs, cs], buf.at[k], sem)

    def drain_in(buf, sem):
        for k in range(K):
            pltpu.make_async_copy(
                scores_hbm.at[0, 0, pl.ds(0, TR), pl.ds(0, TCOL)],
                buf.at[k], sem).wait()

    def fire_out(j, buf, sem):
        b, rs, cs = chunk_slices(j)
        for k in range(K):
            pltpu.async_copy(buf.at[k], out_hbm.at[b, k, rs, cs], sem)

    def drain_out(buf, sem):
        for k in range(K):
            pltpu.make_async_copy(
                buf.at[k],
                out_hbm.at[0, 0, pl.ds(0, TR), pl.ds(0, TCOL)],
                sem).wait()

    def compute(in_buf, out_buf):
        def body(g, unused):
            r = g // (TCOL // LANES)
            c0 = (g - r * (TCOL // LANES)) * LANES
            for k in range(K):
                s_k = in_buf[k, r, pl.ds(c0, LANES)]
                if k == 0:
                    s = [s_k]
                    m = s_k
                else:
                    s.append(s_k)
                    m = jnp.maximum(m, s_k)
            notfound = jnp.ones((LANES,), jnp.float32)
            for k in range(K):
                is_max = jnp.where(s[k] == m, 1.0, 0.0)
                out_buf[k, r, pl.ds(c0, LANES)] = is_max * notfound
                notfound = notfound * (1.0 - is_max)
            return unused

        lax.fori_loop(0, GROUPS, body, 0, unroll=False)

    fire_in(0, in0, si0)

    def step(it, unused):
        j0 = it * 2
        fire_in(j0 + 1, in1, si1)
        drain_in(in0, si0)

        @pl.when(it > 0)
        def _():
            drain_out(out0, so0)

        compute(in0, out0)
        fire_out(j0, out0, so0)

        @pl.when(it < CPW // 2 - 1)
        def _():
            fire_in(j0 + 2, in0, si0)

        drain_in(in1, si1)

        @pl.when(it > 0)
        def _():
            drain_out(out1, so1)

        compute(in1, out1)
        fire_out(j0 + 1, out1, so1)
        return unused

    lax.fori_loop(0, CPW // 2, step, 0, unroll=False)
    drain_out(out0, so0)
    drain_out(out1, so1)


def kernel(candidate_images, selection_scores, mask, partial_image):
    final_image = _final_image(
        candidate_images, selection_scores, mask, partial_image)
    weights = _weights_sc(selection_scores)
    return (final_image, weights)


# speedup vs baseline: 3.2023x; 1.4446x over previous
import jax
import jax.numpy as jnp
from jax.experimental import pallas as pl

B, K, C, H, W = 4, 16, 3, 384, 384
HB = 64


def _selector_block(scores_ref, cand_ref, mask_ref, partial_ref,
                    final_ref, weights_ref):
    scores = scores_ref[0]
    best = jnp.argmax(scores, axis=0).astype(jnp.int32)

    vis = mask_ref[0, 0]
    fill = 1.0 - vis

    sel = [None, None, None]
    for k in range(K):
        onehot = best == k
        weights_ref[0, k] = onehot.astype(jnp.float32)
        for c in range(C):
            pix = jnp.where(onehot, cand_ref[0, k, c], 0.0)
            sel[c] = pix if sel[c] is None else sel[c] + pix

    for c in range(C):
        final_ref[0, c] = partial_ref[0, c] * vis + sel[c] * fill


def kernel(candidate_images, selection_scores, mask, partial_image):
    grid = (B, H // HB)
    final_image, selection_weights = pl.pallas_call(
        _selector_block,
        grid=grid,
        in_specs=[
            pl.BlockSpec((1, K, HB, W), lambda b, h: (b, 0, h, 0)),
            pl.BlockSpec((1, K, C, HB, W), lambda b, h: (b, 0, 0, h, 0)),
            pl.BlockSpec((1, 1, HB, W), lambda b, h: (b, 0, h, 0)),
            pl.BlockSpec((1, C, HB, W), lambda b, h: (b, 0, h, 0)),
        ],
        out_specs=[
            pl.BlockSpec((1, C, HB, W), lambda b, h: (b, 0, h, 0)),
            pl.BlockSpec((1, K, HB, W), lambda b, h: (b, 0, h, 0)),
        ],
        out_shape=[
            jax.ShapeDtypeStruct((B, C, H, W), jnp.float32),
            jax.ShapeDtypeStruct((B, K, H, W), jnp.float32),
        ],
    )(selection_scores, candidate_images, mask, partial_image)
    return (final_image, selection_weights)
